# baseline (speedup 1.0000x reference)
import functools

import jax
import jax.numpy as jnp
from jax import lax
from jax.experimental import pallas as pl
from jax.experimental.pallas import tpu as pltpu
from jax.experimental.pallas import tpu_sc as plsc

VOCAB = 1000000
N_EMBED = 64
BLOCK = 2048
B, T = 4, 2048

_INFO = plsc.get_sparse_core_info()
_L = _INFO.num_lanes
_NW = _INFO.num_cores * _INFO.num_subcores
_ROWS = B * T
_PER_W = _ROWS // _NW
_VECS = N_EMBED // _L
_TCOL = 128
_DEPTH = 6
_AHEAD = _DEPTH - 1


def _sc_embed(table_t, idx_flat, pos):
    mesh = plsc.VectorSubcoreMesh(core_axis_name="c", subcore_axis_name="s")

    @functools.partial(
        pl.kernel,
        mesh=mesh,
        out_type=jax.ShapeDtypeStruct((_ROWS, N_EMBED), jnp.float32),
        compiler_params=pltpu.CompilerParams(
            use_tc_tiling_on_sc=True, needs_layout_passes=False),
        scratch_types=[
            pltpu.VMEM((_PER_W,), jnp.int32),
            pltpu.VMEM((_DEPTH, N_EMBED, _TCOL), jnp.float32),
            pltpu.VMEM((_PER_W, N_EMBED), jnp.float32),
            pltpu.VMEM((_PER_W, N_EMBED), jnp.float32),
            pltpu.SemaphoreType.DMA,
        ],
    )
    def k(table_hbm, idx_hbm, pos_hbm, out_hbm,
          idx_v, chunk_v, rows_v, pos_v, sem):
        wid = lax.axis_index("s") * _INFO.num_cores + lax.axis_index("c")
        row0 = wid * _PER_W
        t0 = (wid % (T // _PER_W)) * _PER_W

        pltpu.sync_copy(idx_hbm.at[pl.ds(row0, _PER_W)], idx_v)
        pltpu.sync_copy(pos_hbm.at[pl.ds(t0, _PER_W)], pos_v)

        lane = lax.iota(jnp.int32, _L)
        nwaves = _PER_W // _L

        def fire(ti, slot):
            q0 = pl.multiple_of((ti // _TCOL) * _TCOL, _TCOL)
            for t in range(N_EMBED // 8):
                pltpu.async_copy(
                    table_hbm.at[pl.ds(8 * t, 8), pl.ds(q0, _TCOL)],
                    chunk_v.at[slot, pl.ds(8 * t, 8)], sem)

        def drain():
            pltpu.make_async_copy(
                table_hbm.at[:, pl.ds(0, _TCOL)], chunk_v.at[0], sem).wait()

        ivec0 = idx_v[pl.ds(0, _L)]
        for j in range(_AHEAD):
            fire(ivec0[j], j)

        def body(g, _):
            ivec = idx_v[pl.ds(g * _L, _L)]
            nvec = idx_v[pl.ds(lax.min(g + 1, nwaves - 1) * _L, _L)]
            for j in range(_L):
                jn = j + _AHEAD
                if jn < _L:
                    fire(ivec[jn], lax.rem(g * _L + jn, _DEPTH))
                else:
                    @pl.when(g + 1 < nwaves)
                    def _():
                        fire(nvec[jn - _L], lax.rem(g * _L + jn, _DEPTH))
                drain()
                r = g * _L + j
                m = lax.rem(ivec[j], _TCOL)
                buf = chunk_v.at[lax.rem(r, _DEPTH)]
                for v in range(_VECS):
                    cvec = plsc.load_gather(buf, [v * _L + lane, m + 0 * lane])
                    sl = pl.ds(v * _L, _L)
                    rows_v[r, sl] = cvec + pos_v[r, sl]
            return 0

        lax.fori_loop(0, nwaves, body, 0)

        pltpu.sync_copy(rows_v, out_hbm.at[pl.ds(row0, _PER_W)])

    return k(table_t, idx_flat, pos)


def kernel(idx, token_table, pos_table):
    idx_flat = idx.astype(jnp.int32).reshape(_ROWS)
    out = _sc_embed(token_table.T, idx_flat, pos_table)
    return out.reshape(B, T, N_EMBED)


# speedup vs baseline: 1.0013x; 1.0013x over previous
import functools

import jax
import jax.numpy as jnp
from jax import lax
from jax.experimental import pallas as pl
from jax.experimental.pallas import tpu as pltpu
from jax.experimental.pallas import tpu_sc as plsc

VOCAB = 1000000
N_EMBED = 64
BLOCK = 2048
B, T = 4, 2048

_INFO = plsc.get_sparse_core_info()
_L = _INFO.num_lanes
_NW = _INFO.num_cores * _INFO.num_subcores
_ROWS = B * T
_PER_W = _ROWS // _NW
_VECS = N_EMBED // _L
_TCOL = 128
_DEPTH = 7
_AHEAD = _DEPTH - 1


def _sc_embed(table_t, idx_flat, pos):
    mesh = plsc.VectorSubcoreMesh(core_axis_name="c", subcore_axis_name="s")

    @functools.partial(
        pl.kernel,
        mesh=mesh,
        out_type=jax.ShapeDtypeStruct((_ROWS, N_EMBED), jnp.float32),
        compiler_params=pltpu.CompilerParams(
            use_tc_tiling_on_sc=True, needs_layout_passes=False),
        scratch_types=[
            pltpu.VMEM((_PER_W,), jnp.int32),
            pltpu.VMEM((_DEPTH, N_EMBED, _TCOL), jnp.float32),
            pltpu.VMEM((_PER_W, N_EMBED), jnp.float32),
            pltpu.VMEM((_PER_W, N_EMBED), jnp.float32),
            pltpu.SemaphoreType.DMA,
        ],
    )
    def k(table_hbm, idx_hbm, pos_hbm, out_hbm,
          idx_v, chunk_v, rows_v, pos_v, sem):
        wid = lax.axis_index("s") * _INFO.num_cores + lax.axis_index("c")
        row0 = wid * _PER_W
        t0 = (wid % (T // _PER_W)) * _PER_W

        pltpu.sync_copy(idx_hbm.at[pl.ds(row0, _PER_W)], idx_v)
        pltpu.sync_copy(pos_hbm.at[pl.ds(t0, _PER_W)], pos_v)

        lane = lax.iota(jnp.int32, _L)
        nwaves = _PER_W // _L

        def fire(ti, slot):
            q0 = (ti // _TCOL) * _TCOL
            pltpu.async_copy(
                table_hbm.at[:, pl.ds(pl.multiple_of(q0, _TCOL), _TCOL)],
                chunk_v.at[slot], sem)

        def drain():
            pltpu.make_async_copy(
                table_hbm.at[:, pl.ds(0, _TCOL)], chunk_v.at[0], sem).wait()

        ivec0 = idx_v[pl.ds(0, _L)]
        for j in range(_AHEAD):
            fire(ivec0[j], j)

        def body(g, _):
            ivec = idx_v[pl.ds(g * _L, _L)]
            nvec = idx_v[pl.ds(lax.min(g + 1, nwaves - 1) * _L, _L)]
            for j in range(_L):
                jn = j + _AHEAD
                if jn < _L:
                    fire(ivec[jn], lax.rem(g * _L + jn, _DEPTH))
                else:
                    @pl.when(g + 1 < nwaves)
                    def _():
                        fire(nvec[jn - _L], lax.rem(g * _L + jn, _DEPTH))
                drain()
                r = g * _L + j
                m = lax.rem(ivec[j], _TCOL)
                buf = chunk_v.at[lax.rem(r, _DEPTH)]
                for v in range(_VECS):
                    cvec = plsc.load_gather(buf, [v * _L + lane, m + 0 * lane])
                    sl = pl.ds(v * _L, _L)
                    rows_v[r, sl] = cvec + pos_v[r, sl]
            return 0

        lax.fori_loop(0, nwaves, body, 0)

        pltpu.sync_copy(rows_v, out_hbm.at[pl.ds(row0, _PER_W)])

    return k(table_t, idx_flat, pos)


def kernel(idx, token_table, pos_table):
    idx_flat = idx.astype(jnp.int32).reshape(_ROWS)
    out = _sc_embed(token_table.T, idx_flat, pos_table)
    return out.reshape(B, T, N_EMBED)


# speedup vs baseline: 1.0084x; 1.0071x over previous
import functools

import jax
import jax.numpy as jnp
from jax import lax
from jax.experimental import pallas as pl
from jax.experimental.pallas import tpu as pltpu
from jax.experimental.pallas import tpu_sc as plsc

VOCAB = 1000000
N_EMBED = 64
BLOCK = 2048
B, T = 4, 2048

_INFO = plsc.get_sparse_core_info()
_L = _INFO.num_lanes
_NW = _INFO.num_cores * _INFO.num_subcores
_ROWS = B * T
_PER_W = _ROWS // _NW
_VECS = N_EMBED // _L
_TCOL = 128
_DEPTH = 6
_AHEAD = _DEPTH - 1


def _sc_embed(table_t, idx_flat, pos):
    mesh = plsc.VectorSubcoreMesh(core_axis_name="c", subcore_axis_name="s")

    @functools.partial(
        pl.kernel,
        mesh=mesh,
        out_type=jax.ShapeDtypeStruct((_ROWS, N_EMBED), jnp.float32),
        compiler_params=pltpu.CompilerParams(
            use_tc_tiling_on_sc=True, needs_layout_passes=False),
        scratch_types=[
            pltpu.VMEM((_PER_W,), jnp.int32),
            pltpu.VMEM((_DEPTH, N_EMBED, _TCOL), jnp.float32),
            pltpu.VMEM((_PER_W, N_EMBED), jnp.float32),
            pltpu.VMEM((_PER_W, N_EMBED), jnp.float32),
            pltpu.SemaphoreType.DMA,
        ],
    )
    def k(table_hbm, idx_hbm, pos_hbm, out_hbm,
          idx_v, chunk_v, rows_v, pos_v, sem):
        wid = lax.axis_index("s") * _INFO.num_cores + lax.axis_index("c")
        row0 = wid * _PER_W
        t0 = (wid % (T // _PER_W)) * _PER_W

        pltpu.sync_copy(idx_hbm.at[pl.ds(row0, _PER_W)], idx_v)
        pltpu.sync_copy(pos_hbm.at[pl.ds(t0, _PER_W)], pos_v)

        lane = lax.iota(jnp.int32, _L)
        nwaves = _PER_W // _L

        def fire(ti, slot):
            q0 = (ti // _TCOL) * _TCOL
            pltpu.async_copy(
                table_hbm.at[:, pl.ds(pl.multiple_of(q0, _TCOL), _TCOL)],
                chunk_v.at[slot], sem)

        def drain():
            pltpu.make_async_copy(
                table_hbm.at[:, pl.ds(0, _TCOL)], chunk_v.at[0], sem).wait()

        ivec0 = idx_v[pl.ds(0, _L)]
        for j in range(_AHEAD):
            fire(ivec0[j], j)

        def body(g, _):
            ivec = idx_v[pl.ds(g * _L, _L)]
            nvec = idx_v[pl.ds(lax.min(g + 1, nwaves - 1) * _L, _L)]
            for j in range(_L):
                jn = j + _AHEAD
                if jn < _L:
                    fire(ivec[jn], lax.rem(g * _L + jn, _DEPTH))
                else:
                    @pl.when(g + 1 < nwaves)
                    def _():
                        fire(nvec[jn - _L], lax.rem(g * _L + jn, _DEPTH))
                drain()
                r = g * _L + j
                m = lax.rem(ivec[j], _TCOL)
                buf = chunk_v.at[lax.rem(r, _DEPTH)]
                for v in range(_VECS):
                    cvec = plsc.load_gather(buf, [v * _L + lane, m + 0 * lane])
                    sl = pl.ds(v * _L, _L)
                    rows_v[r, sl] = cvec + pos_v[r, sl]
            return 0

        lax.fori_loop(0, nwaves, body, 0)

        pltpu.sync_copy(rows_v, out_hbm.at[pl.ds(row0, _PER_W)])

    return k(table_t, idx_flat, pos)


def kernel(idx, token_table, pos_table):
    idx_flat = idx.astype(jnp.int32).reshape(_ROWS)
    out = _sc_embed(token_table.T, idx_flat, pos_table)
    return out.reshape(B, T, N_EMBED)
